# baseline (speedup 1.0000x reference)
import functools

import jax
import jax.numpy as jnp
from jax import lax
from jax.experimental import pallas as pl
from jax.experimental.pallas import tpu as pltpu
from jax.experimental.pallas import tpu_sc as plsc

TILE = 256
_NC, _NS = 2, 16
_NW = _NC * _NS


def _router_body(x_ref, wg_ref, w1o_ref, w2o_ref, e1o_ref, e2o_ref, *, ne):
    x = x_ref[...]
    logits = lax.dot_general(x, wg_ref[...], (((1,), (1,)), ((), ())),
                             preferred_element_type=jnp.float32)
    m = jnp.max(logits, axis=1, keepdims=True)
    ex = jnp.exp(logits - m)
    p = ex / jnp.sum(ex, axis=1, keepdims=True)
    iota = lax.broadcasted_iota(jnp.int32, p.shape, 1)
    v1 = jnp.max(p, axis=1, keepdims=True)
    e1 = jnp.min(jnp.where(p >= v1, iota, ne), axis=1, keepdims=True)
    pm = jnp.where(iota == e1, -1.0, p)
    v2 = jnp.max(pm, axis=1, keepdims=True)
    e2 = jnp.min(jnp.where(pm >= v2, iota, ne), axis=1, keepdims=True)
    s = v1 + v2
    w1o_ref[...] = v1 / s
    w2o_ref[...] = v2 / s
    e1o_ref[...] = e1
    e2o_ref[...] = e2


def _router(x, wg, *, interpret=False):
    t, d = x.shape
    ne = wg.shape[0]
    rows = min(1024, t)
    grid = (t // rows,)
    body = functools.partial(_router_body, ne=ne)
    return pl.pallas_call(
        body,
        grid=grid,
        in_specs=[
            pl.BlockSpec((rows, d), lambda r: (r, 0)),
            pl.BlockSpec((ne, d), lambda r: (0, 0)),
        ],
        out_specs=[
            pl.BlockSpec((rows, 1), lambda r: (r, 0)),
            pl.BlockSpec((rows, 1), lambda r: (r, 0)),
            pl.BlockSpec((rows, 1), lambda r: (r, 0)),
            pl.BlockSpec((rows, 1), lambda r: (r, 0)),
        ],
        out_shape=[
            jax.ShapeDtypeStruct((t, 1), jnp.float32),
            jax.ShapeDtypeStruct((t, 1), jnp.float32),
            jax.ShapeDtypeStruct((t, 1), jnp.int32),
            jax.ShapeDtypeStruct((t, 1), jnp.int32),
        ],
        interpret=interpret,
    )(x, wg)


def _scan_rows(c, t):
    k = 1
    while k < t:
        pad = jnp.zeros((k,) + c.shape[1:], c.dtype)
        c = c + jnp.concatenate([pad, c[:-k]], axis=0)
        k *= 2
    return c


def _scan_lanes(c, n):
    k = 1
    while k < n:
        pad = jnp.zeros(c.shape[:1] + (k,), c.dtype)
        c = c + jnp.concatenate([pad, c[:, :-k]], axis=1)
        k *= 2
    return c


def _plan_body(e1_ref, e2_ref, dest_ref, te_ref, *, t, ne, nt):
    e1 = e1_ref[...]
    e2 = e2_ref[...]
    io_e1 = lax.broadcasted_iota(jnp.int32, (t, ne), 1)
    oh1 = (e1 == io_e1).astype(jnp.int32)
    oh2 = (e2 == io_e1).astype(jnp.int32)
    c1 = _scan_rows(oh1, t)
    c2 = _scan_rows(oh2, t)
    ex1 = c1 - oh1
    ex2 = c2 - oh2
    cnt1 = c1[t - 1:t, :]
    cnt = cnt1 + c2[t - 1:t, :]
    pc = ((cnt + (TILE - 1)) // TILE) * TILE
    off = _scan_lanes(pc, ne) - pc
    rank1 = jnp.sum(oh1 * ex1, axis=1, keepdims=True)
    rank2 = jnp.sum(oh2 * (cnt1 + ex2), axis=1, keepdims=True)
    base1 = jnp.sum(oh1 * off, axis=1, keepdims=True)
    base2 = jnp.sum(oh2 * off, axis=1, keepdims=True)
    dest_ref[...] = jnp.concatenate([base1 + rank1, base2 + rank2], axis=0)
    endc = off + pc
    tid = lax.broadcasted_iota(jnp.int32, (nt + 1, ne), 0)
    te = jnp.sum((tid * TILE >= endc).astype(jnp.int32), axis=1, keepdims=True)
    nlive = jnp.sum(pc, axis=1, keepdims=True) // TILE
    te_map = jnp.minimum(te[:nt], ne - 1)
    te_ref[...] = jnp.concatenate([te_map, nlive], axis=0)


def _plan(e1, e2, ne, nt, *, interpret=False):
    t = e1.shape[0]
    body = functools.partial(_plan_body, t=t, ne=ne, nt=nt)
    return pl.pallas_call(
        body,
        out_shape=[
            jax.ShapeDtypeStruct((2 * t, 1), jnp.int32),
            jax.ShapeDtypeStruct((nt + 1, 1), jnp.int32),
        ],
        interpret=interpret,
    )(e1, e2)


_CHUNK = 32


def _sc_scatter_rows(x, dest3, np_rows):
    t, d = x.shape
    nch = dest3.shape[1]
    per_w = nch * _CHUNK
    mesh = plsc.VectorSubcoreMesh(core_axis_name="c", subcore_axis_name="s")

    @functools.partial(
        pl.kernel, mesh=mesh,
        out_type=jax.ShapeDtypeStruct((np_rows, d), jnp.float32),
        scratch_types=[
            pltpu.VMEM((nch, _CHUNK), jnp.int32),
            pltpu.VMEM((2, _CHUNK, d), jnp.float32),
            pltpu.SemaphoreType.DMA,
            pltpu.SemaphoreType.DMA,
            pltpu.SemaphoreType.DMA,
            pltpu.SemaphoreType.DMA,
        ],
    )
    def k(x_hbm, dest_hbm, xs_hbm, idx_all, rows2, sr0, sr1, ss0, ss1):
        wid = lax.axis_index("s") * _NC + lax.axis_index("c")
        pltpu.sync_copy(dest_hbm.at[wid], idx_all)
        semr = (sr0, sr1)
        sems = (ss0, ss1)
        reads = [None, None]
        scats = [None, None]

        def issue_read(c):
            b = c % 2
            j0 = wid * per_w + c * _CHUNK
            r0 = lax.rem(j0, t)
            reads[b] = pltpu.async_copy(
                x_hbm.at[pl.ds(r0, _CHUNK)], rows2.at[b], semr[b])

        issue_read(0)
        for c in range(nch):
            b = c % 2
            if c + 1 < nch:
                if scats[(c + 1) % 2] is not None:
                    scats[(c + 1) % 2].wait()
                    scats[(c + 1) % 2] = None
                issue_read(c + 1)
            reads[b].wait()
            scats[b] = pltpu.async_copy(
                rows2.at[b], xs_hbm.at[idx_all.at[c]], sems[b])
        for b in range(2):
            if scats[b] is not None:
                scats[b].wait()

    return k(x, dest3)


def _sc_gather_rows(ys, dest3, n_tasks):
    d = ys.shape[1]
    nch = dest3.shape[1]
    per_w = nch * _CHUNK
    mesh = plsc.VectorSubcoreMesh(core_axis_name="c", subcore_axis_name="s")

    @functools.partial(
        pl.kernel, mesh=mesh,
        out_type=jax.ShapeDtypeStruct((n_tasks, d), jnp.float32),
        scratch_types=[
            pltpu.VMEM((nch, _CHUNK), jnp.int32),
            pltpu.VMEM((2, _CHUNK, d), jnp.float32),
            pltpu.SemaphoreType.DMA,
            pltpu.SemaphoreType.DMA,
            pltpu.SemaphoreType.DMA,
            pltpu.SemaphoreType.DMA,
        ],
    )
    def k(ys_hbm, dest_hbm, yg_hbm, idx_all, rows2, sg0, sg1, sw0, sw1):
        wid = lax.axis_index("s") * _NC + lax.axis_index("c")
        pltpu.sync_copy(dest_hbm.at[wid], idx_all)
        semg = (sg0, sg1)
        semw = (sw0, sw1)
        gats = [None, None]
        wrs = [None, None]

        def issue_gather(c):
            b = c % 2
            gats[b] = pltpu.async_copy(
                ys_hbm.at[idx_all.at[c]], rows2.at[b], semg[b])

        issue_gather(0)
        for c in range(nch):
            b = c % 2
            if c + 1 < nch:
                if wrs[(c + 1) % 2] is not None:
                    wrs[(c + 1) % 2].wait()
                    wrs[(c + 1) % 2] = None
                issue_gather(c + 1)
            gats[b].wait()
            j0 = wid * per_w + c * _CHUNK
            wrs[b] = pltpu.async_copy(
                rows2.at[b], yg_hbm.at[pl.ds(j0, _CHUNK)], semw[b])
        for b in range(2):
            if wrs[b] is not None:
                wrs[b].wait()

    return k(ys, dest3)


def _gmlp_body(te_ref, xs_ref, w1_ref, w3_ref, w2_ref, ys_ref, *, nt):
    i = pl.program_id(0)

    @pl.when(i < te_ref[nt])
    def _compute():
        h = xs_ref[...]
        a = lax.dot_general(h, w1_ref[0], (((1,), (1,)), ((), ())),
                            preferred_element_type=jnp.float32)
        a = a * (1.0 / (1.0 + jnp.exp(-a)))
        b = lax.dot_general(h, w3_ref[0], (((1,), (1,)), ((), ())),
                            preferred_element_type=jnp.float32)
        y = lax.dot_general(a * b, w2_ref[0], (((1,), (1,)), ((), ())),
                            preferred_element_type=jnp.float32)
        ys_ref[...] = y


def _gmlp(te, xs, w1, w3, w2, *, interpret=False):
    np_rows, d = xs.shape
    ne, ff, _ = w1.shape
    nt = np_rows // TILE
    body = functools.partial(_gmlp_body, nt=nt)
    grid_spec = pltpu.PrefetchScalarGridSpec(
        num_scalar_prefetch=1,
        grid=(nt,),
        in_specs=[
            pl.BlockSpec((TILE, d), lambda i, te_r: (i, 0)),
            pl.BlockSpec((1, ff, d), lambda i, te_r: (te_r[i], 0, 0)),
            pl.BlockSpec((1, ff, d), lambda i, te_r: (te_r[i], 0, 0)),
            pl.BlockSpec((1, d, ff), lambda i, te_r: (te_r[i], 0, 0)),
        ],
        out_specs=pl.BlockSpec((TILE, d), lambda i, te_r: (i, 0)),
    )
    return pl.pallas_call(
        body,
        grid_spec=grid_spec,
        out_shape=jax.ShapeDtypeStruct((np_rows, d), jnp.float32),
        compiler_params=pltpu.CompilerParams(
            dimension_semantics=("arbitrary",),
        ),
        interpret=interpret,
    )(te, xs, w1, w3, w2)


def _combine_body(w1_ref, w2_ref, y1_ref, y2_ref, out_ref):
    out_ref[...] = y1_ref[...] * w1_ref[...] + y2_ref[...] * w2_ref[...]


def _combine(w1n, w2n, y1, y2, *, interpret=False):
    t, d = y1.shape
    rows = min(1024, t)
    return pl.pallas_call(
        _combine_body,
        grid=(t // rows,),
        in_specs=[
            pl.BlockSpec((rows, 1), lambda r: (r, 0)),
            pl.BlockSpec((rows, 1), lambda r: (r, 0)),
            pl.BlockSpec((rows, d), lambda r: (r, 0)),
            pl.BlockSpec((rows, d), lambda r: (r, 0)),
        ],
        out_specs=pl.BlockSpec((rows, d), lambda r: (r, 0)),
        out_shape=jax.ShapeDtypeStruct((t, d), jnp.float32),
        interpret=interpret,
    )(w1n, w2n, y1, y2)


def kernel(hidden_states, Wg, W1, W3, W2):
    bsz, seq, d = hidden_states.shape
    ne = Wg.shape[0]
    x = hidden_states.reshape(-1, d)
    t = x.shape[0]
    np_rows = 2 * t + ne * TILE
    nt = np_rows // TILE

    w1n, w2n, e1, e2 = _router(x, Wg)
    dest, te = _plan(e1, e2, ne, nt)
    dest3 = dest.reshape(_NW, (2 * t) // (_NW * _CHUNK), _CHUNK)
    xs = _sc_scatter_rows(x, dest3, np_rows)
    ys = _gmlp(te.reshape(-1), xs, W1, W3, W2)
    yg = _sc_gather_rows(ys, dest3, 2 * t)
    out = _combine(w1n, w2n, yg[:t], yg[t:])
    return out.reshape(bsz, seq, d)


# speedup vs baseline: 1.0742x; 1.0742x over previous
import functools

import jax
import jax.numpy as jnp
from jax import lax
from jax.experimental import pallas as pl
from jax.experimental.pallas import tpu as pltpu
from jax.experimental.pallas import tpu_sc as plsc

TILE = 256
_NC, _NS = 2, 16
_NW = _NC * _NS


def _router_body(x_ref, wg_ref, w1o_ref, w2o_ref, e1o_ref, e2o_ref, *, ne):
    x = x_ref[...]
    logits = lax.dot_general(x, wg_ref[...], (((1,), (1,)), ((), ())),
                             preferred_element_type=jnp.float32)
    m = jnp.max(logits, axis=1, keepdims=True)
    ex = jnp.exp(logits - m)
    p = ex / jnp.sum(ex, axis=1, keepdims=True)
    iota = lax.broadcasted_iota(jnp.int32, p.shape, 1)
    v1 = jnp.max(p, axis=1, keepdims=True)
    e1 = jnp.min(jnp.where(p >= v1, iota, ne), axis=1, keepdims=True)
    pm = jnp.where(iota == e1, -1.0, p)
    v2 = jnp.max(pm, axis=1, keepdims=True)
    e2 = jnp.min(jnp.where(pm >= v2, iota, ne), axis=1, keepdims=True)
    s = v1 + v2
    w1o_ref[...] = v1 / s
    w2o_ref[...] = v2 / s
    e1o_ref[...] = e1
    e2o_ref[...] = e2


def _router(x, wg, *, interpret=False):
    t, d = x.shape
    ne = wg.shape[0]
    rows = min(1024, t)
    grid = (t // rows,)
    body = functools.partial(_router_body, ne=ne)
    return pl.pallas_call(
        body,
        grid=grid,
        in_specs=[
            pl.BlockSpec((rows, d), lambda r: (r, 0)),
            pl.BlockSpec((ne, d), lambda r: (0, 0)),
        ],
        out_specs=[
            pl.BlockSpec((rows, 1), lambda r: (r, 0)),
            pl.BlockSpec((rows, 1), lambda r: (r, 0)),
            pl.BlockSpec((rows, 1), lambda r: (r, 0)),
            pl.BlockSpec((rows, 1), lambda r: (r, 0)),
        ],
        out_shape=[
            jax.ShapeDtypeStruct((t, 1), jnp.float32),
            jax.ShapeDtypeStruct((t, 1), jnp.float32),
            jax.ShapeDtypeStruct((t, 1), jnp.int32),
            jax.ShapeDtypeStruct((t, 1), jnp.int32),
        ],
        interpret=interpret,
    )(x, wg)


def _scan_rows(c, t):
    k = 1
    while k < t:
        pad = jnp.zeros((k,) + c.shape[1:], c.dtype)
        c = c + jnp.concatenate([pad, c[:-k]], axis=0)
        k *= 2
    return c


def _scan_lanes(c, n):
    k = 1
    while k < n:
        pad = jnp.zeros(c.shape[:1] + (k,), c.dtype)
        c = c + jnp.concatenate([pad, c[:, :-k]], axis=1)
        k *= 2
    return c


def _plan_body(e1_ref, e2_ref, dest_ref, te_ref, *, t, ne, nt):
    e1 = e1_ref[...]
    e2 = e2_ref[...]
    io_e1 = lax.broadcasted_iota(jnp.int32, (t, ne), 1)
    oh1 = (e1 == io_e1).astype(jnp.int32)
    oh2 = (e2 == io_e1).astype(jnp.int32)
    c1 = _scan_rows(oh1, t)
    c2 = _scan_rows(oh2, t)
    ex1 = c1 - oh1
    ex2 = c2 - oh2
    cnt1 = c1[t - 1:t, :]
    cnt = cnt1 + c2[t - 1:t, :]
    pc = ((cnt + (TILE - 1)) // TILE) * TILE
    off = _scan_lanes(pc, ne) - pc
    rank1 = jnp.sum(oh1 * ex1, axis=1, keepdims=True)
    rank2 = jnp.sum(oh2 * (cnt1 + ex2), axis=1, keepdims=True)
    base1 = jnp.sum(oh1 * off, axis=1, keepdims=True)
    base2 = jnp.sum(oh2 * off, axis=1, keepdims=True)
    dest_ref[...] = jnp.concatenate([base1 + rank1, base2 + rank2], axis=0)
    endc = off + pc
    tid = lax.broadcasted_iota(jnp.int32, (nt + 1, ne), 0)
    te = jnp.sum((tid * TILE >= endc).astype(jnp.int32), axis=1, keepdims=True)
    nlive = jnp.sum(pc, axis=1, keepdims=True) // TILE
    te_map = jnp.minimum(te[:nt], ne - 1)
    te_ref[...] = jnp.concatenate([te_map, nlive], axis=0)


def _plan(e1, e2, ne, nt, *, interpret=False):
    t = e1.shape[0]
    body = functools.partial(_plan_body, t=t, ne=ne, nt=nt)
    return pl.pallas_call(
        body,
        out_shape=[
            jax.ShapeDtypeStruct((2 * t, 1), jnp.int32),
            jax.ShapeDtypeStruct((nt + 1, 1), jnp.int32),
        ],
        interpret=interpret,
    )(e1, e2)


_CHUNK = 32


def _sc_scatter_rows(x, dest3, np_rows):
    t, d = x.shape
    nch = dest3.shape[1]
    per_w = nch * _CHUNK
    mesh = plsc.VectorSubcoreMesh(core_axis_name="c", subcore_axis_name="s")

    @functools.partial(
        pl.kernel, mesh=mesh,
        out_type=jax.ShapeDtypeStruct((np_rows, d), jnp.float32),
        scratch_types=[
            pltpu.VMEM((nch, _CHUNK), jnp.int32),
            pltpu.VMEM((2, _CHUNK, d), jnp.float32),
            pltpu.SemaphoreType.DMA,
            pltpu.SemaphoreType.DMA,
            pltpu.SemaphoreType.DMA,
            pltpu.SemaphoreType.DMA,
        ],
    )
    def k(x_hbm, dest_hbm, xs_hbm, idx_all, rows2, sr0, sr1, ss0, ss1):
        wid = lax.axis_index("s") * _NC + lax.axis_index("c")
        pltpu.sync_copy(dest_hbm.at[wid], idx_all)
        semr = (sr0, sr1)
        sems = (ss0, ss1)
        reads = [None, None]
        scats = [None, None]

        def issue_read(c):
            b = c % 2
            j0 = wid * per_w + c * _CHUNK
            r0 = lax.rem(j0, t)
            reads[b] = pltpu.async_copy(
                x_hbm.at[pl.ds(r0, _CHUNK)], rows2.at[b], semr[b])

        issue_read(0)
        for c in range(nch):
            b = c % 2
            if c + 1 < nch:
                if scats[(c + 1) % 2] is not None:
                    scats[(c + 1) % 2].wait()
                    scats[(c + 1) % 2] = None
                issue_read(c + 1)
            reads[b].wait()
            scats[b] = pltpu.async_copy(
                rows2.at[b], xs_hbm.at[idx_all.at[c]], sems[b])
        for b in range(2):
            if scats[b] is not None:
                scats[b].wait()

    return k(x, dest3)


def _sc_gather_rows(ys, dest3, n_tasks):
    d = ys.shape[1]
    nch = dest3.shape[1]
    per_w = nch * _CHUNK
    mesh = plsc.VectorSubcoreMesh(core_axis_name="c", subcore_axis_name="s")

    @functools.partial(
        pl.kernel, mesh=mesh,
        out_type=jax.ShapeDtypeStruct((n_tasks, d), jnp.float32),
        scratch_types=[
            pltpu.VMEM((nch, _CHUNK), jnp.int32),
            pltpu.VMEM((2, _CHUNK, d), jnp.float32),
            pltpu.SemaphoreType.DMA,
            pltpu.SemaphoreType.DMA,
            pltpu.SemaphoreType.DMA,
            pltpu.SemaphoreType.DMA,
        ],
    )
    def k(ys_hbm, dest_hbm, yg_hbm, idx_all, rows2, sg0, sg1, sw0, sw1):
        wid = lax.axis_index("s") * _NC + lax.axis_index("c")
        pltpu.sync_copy(dest_hbm.at[wid], idx_all)
        semg = (sg0, sg1)
        semw = (sw0, sw1)
        gats = [None, None]
        wrs = [None, None]

        def issue_gather(c):
            b = c % 2
            gats[b] = pltpu.async_copy(
                ys_hbm.at[idx_all.at[c]], rows2.at[b], semg[b])

        issue_gather(0)
        for c in range(nch):
            b = c % 2
            if c + 1 < nch:
                if wrs[(c + 1) % 2] is not None:
                    wrs[(c + 1) % 2].wait()
                    wrs[(c + 1) % 2] = None
                issue_gather(c + 1)
            gats[b].wait()
            j0 = wid * per_w + c * _CHUNK
            wrs[b] = pltpu.async_copy(
                rows2.at[b], yg_hbm.at[pl.ds(j0, _CHUNK)], semw[b])
        for b in range(2):
            if wrs[b] is not None:
                wrs[b].wait()

    return k(ys, dest3)


def _gmlp_body(te_ref, xs_ref, w1_ref, w3_ref, w2_ref, ys_ref, *, nt):
    i = pl.program_id(0)

    @pl.when(i < te_ref[nt])
    def _compute():
        h = xs_ref[...]
        a = lax.dot_general(h, w1_ref[0], (((1,), (1,)), ((), ())),
                            preferred_element_type=jnp.float32)
        a = a * (1.0 / (1.0 + jnp.exp(-a)))
        b = lax.dot_general(h, w3_ref[0], (((1,), (1,)), ((), ())),
                            preferred_element_type=jnp.float32)
        y = lax.dot_general(a * b, w2_ref[0], (((1,), (1,)), ((), ())),
                            preferred_element_type=jnp.float32)
        ys_ref[...] = y


def _gmlp(te, xs, w1, w3, w2, *, interpret=False):
    np_rows, d = xs.shape
    ne, ff, _ = w1.shape
    nt = np_rows // TILE
    body = functools.partial(_gmlp_body, nt=nt)
    grid_spec = pltpu.PrefetchScalarGridSpec(
        num_scalar_prefetch=1,
        grid=(nt,),
        in_specs=[
            pl.BlockSpec((TILE, d), lambda i, te_r: (i, 0)),
            pl.BlockSpec((1, ff, d), lambda i, te_r: (te_r[i], 0, 0)),
            pl.BlockSpec((1, ff, d), lambda i, te_r: (te_r[i], 0, 0)),
            pl.BlockSpec((1, d, ff), lambda i, te_r: (te_r[i], 0, 0)),
        ],
        out_specs=pl.BlockSpec((TILE, d), lambda i, te_r: (i, 0)),
    )
    return pl.pallas_call(
        body,
        grid_spec=grid_spec,
        out_shape=jax.ShapeDtypeStruct((np_rows, d), jnp.float32),
        compiler_params=pltpu.CompilerParams(
            dimension_semantics=("arbitrary",),
        ),
        interpret=interpret,
    )(te, xs, w1, w3, w2)


def _combine_body(w1_ref, w2_ref, y1_ref, y2_ref, out_ref):
    out_ref[...] = y1_ref[...] * w1_ref[...] + y2_ref[...] * w2_ref[...]


def _combine(w1n, w2n, yg, t, *, interpret=False):
    d = yg.shape[1]
    rows = min(1024, t)
    nb = t // rows
    return pl.pallas_call(
        _combine_body,
        grid=(nb,),
        in_specs=[
            pl.BlockSpec((rows, 1), lambda r: (r, 0)),
            pl.BlockSpec((rows, 1), lambda r: (r, 0)),
            pl.BlockSpec((rows, d), lambda r: (r, 0)),
            pl.BlockSpec((rows, d), lambda r, _nb=nb: (_nb + r, 0)),
        ],
        out_specs=pl.BlockSpec((rows, d), lambda r: (r, 0)),
        out_shape=jax.ShapeDtypeStruct((t, d), jnp.float32),
        interpret=interpret,
    )(w1n, w2n, yg, yg)


def kernel(hidden_states, Wg, W1, W3, W2):
    bsz, seq, d = hidden_states.shape
    ne = Wg.shape[0]
    x = hidden_states.reshape(-1, d)
    t = x.shape[0]
    np_rows = 2 * t + ne * TILE
    nt = np_rows // TILE

    w1n, w2n, e1, e2 = _router(x, Wg)
    dest, te = _plan(e1, e2, ne, nt)
    dest3 = dest.reshape(_NW, (2 * t) // (_NW * _CHUNK), _CHUNK)
    xs = _sc_scatter_rows(x, dest3, np_rows)
    ys = _gmlp(te.reshape(-1), xs, W1, W3, W2)
    yg = _sc_gather_rows(ys, dest3, 2 * t)
    out = _combine(w1n, w2n, yg, t)
    return out.reshape(bsz, seq, d)


# speedup vs baseline: 1.0882x; 1.0131x over previous
import functools

import jax
import jax.numpy as jnp
from jax import lax
from jax.experimental import pallas as pl
from jax.experimental.pallas import tpu as pltpu
from jax.experimental.pallas import tpu_sc as plsc

TILE = 256
_NC, _NS = 2, 16
_NW = _NC * _NS


def _router_body(x_ref, wg_ref, w1o_ref, w2o_ref, e1o_ref, e2o_ref, *, ne):
    x = x_ref[...]
    logits = lax.dot_general(x, wg_ref[...], (((1,), (1,)), ((), ())),
                             preferred_element_type=jnp.float32)
    m = jnp.max(logits, axis=1, keepdims=True)
    ex = jnp.exp(logits - m)
    p = ex / jnp.sum(ex, axis=1, keepdims=True)
    iota = lax.broadcasted_iota(jnp.int32, p.shape, 1)
    v1 = jnp.max(p, axis=1, keepdims=True)
    e1 = jnp.min(jnp.where(p >= v1, iota, ne), axis=1, keepdims=True)
    pm = jnp.where(iota == e1, -1.0, p)
    v2 = jnp.max(pm, axis=1, keepdims=True)
    e2 = jnp.min(jnp.where(pm >= v2, iota, ne), axis=1, keepdims=True)
    s = v1 + v2
    w1o_ref[...] = v1 / s
    w2o_ref[...] = v2 / s
    e1o_ref[...] = e1
    e2o_ref[...] = e2


def _router(x, wg, *, interpret=False):
    t, d = x.shape
    ne = wg.shape[0]
    rows = min(1024, t)
    grid = (t // rows,)
    body = functools.partial(_router_body, ne=ne)
    return pl.pallas_call(
        body,
        grid=grid,
        in_specs=[
            pl.BlockSpec((rows, d), lambda r: (r, 0)),
            pl.BlockSpec((ne, d), lambda r: (0, 0)),
        ],
        out_specs=[
            pl.BlockSpec((rows, 1), lambda r: (r, 0)),
            pl.BlockSpec((rows, 1), lambda r: (r, 0)),
            pl.BlockSpec((rows, 1), lambda r: (r, 0)),
            pl.BlockSpec((rows, 1), lambda r: (r, 0)),
        ],
        out_shape=[
            jax.ShapeDtypeStruct((t, 1), jnp.float32),
            jax.ShapeDtypeStruct((t, 1), jnp.float32),
            jax.ShapeDtypeStruct((t, 1), jnp.int32),
            jax.ShapeDtypeStruct((t, 1), jnp.int32),
        ],
        interpret=interpret,
    )(x, wg)


def _scan_rows(c, t):
    k = 1
    while k < t:
        pad = jnp.zeros((k,) + c.shape[1:], c.dtype)
        c = c + jnp.concatenate([pad, c[:-k]], axis=0)
        k *= 2
    return c


def _scan_lanes(c, n):
    k = 1
    while k < n:
        pad = jnp.zeros(c.shape[:1] + (k,), c.dtype)
        c = c + jnp.concatenate([pad, c[:, :-k]], axis=1)
        k *= 2
    return c


def _route_plan_body(x_ref, wg_ref, w1o_ref, w2o_ref, dest_ref, te_ref,
                     *, t, ne, nt):
    x = x_ref[...]
    logits = lax.dot_general(x, wg_ref[...], (((1,), (1,)), ((), ())),
                             preferred_element_type=jnp.float32)
    m = jnp.max(logits, axis=1, keepdims=True)
    ex = jnp.exp(logits - m)
    p = ex / jnp.sum(ex, axis=1, keepdims=True)
    iota = lax.broadcasted_iota(jnp.int32, p.shape, 1)
    v1 = jnp.max(p, axis=1, keepdims=True)
    e1 = jnp.min(jnp.where(p >= v1, iota, ne), axis=1, keepdims=True)
    pm = jnp.where(iota == e1, -1.0, p)
    v2 = jnp.max(pm, axis=1, keepdims=True)
    e2 = jnp.min(jnp.where(pm >= v2, iota, ne), axis=1, keepdims=True)
    s = v1 + v2
    w1o_ref[...] = v1 / s
    w2o_ref[...] = v2 / s
    io_e1 = lax.broadcasted_iota(jnp.int32, (t, ne), 1)
    oh1 = (e1 == io_e1).astype(jnp.int32)
    oh2 = (e2 == io_e1).astype(jnp.int32)
    c1 = _scan_rows(oh1, t)
    c2 = _scan_rows(oh2, t)
    ex1 = c1 - oh1
    ex2 = c2 - oh2
    cnt1 = c1[t - 1:t, :]
    cnt = cnt1 + c2[t - 1:t, :]
    pc = ((cnt + (TILE - 1)) // TILE) * TILE
    off = _scan_lanes(pc, ne) - pc
    rank1 = jnp.sum(oh1 * ex1, axis=1, keepdims=True)
    rank2 = jnp.sum(oh2 * (cnt1 + ex2), axis=1, keepdims=True)
    base1 = jnp.sum(oh1 * off, axis=1, keepdims=True)
    base2 = jnp.sum(oh2 * off, axis=1, keepdims=True)
    dest_ref[...] = jnp.concatenate([base1 + rank1, base2 + rank2], axis=0)
    endc = off + pc
    tid = lax.broadcasted_iota(jnp.int32, (nt + 1, ne), 0)
    te = jnp.sum((tid * TILE >= endc).astype(jnp.int32), axis=1, keepdims=True)
    nlive = jnp.sum(pc, axis=1, keepdims=True) // TILE
    te_map = jnp.minimum(te[:nt], ne - 1)
    te_ref[...] = jnp.concatenate([te_map, nlive], axis=0)


def _route_plan(x, wg, nt, *, interpret=False):
    t, d = x.shape
    ne = wg.shape[0]
    body = functools.partial(_route_plan_body, t=t, ne=ne, nt=nt)
    return pl.pallas_call(
        body,
        out_shape=[
            jax.ShapeDtypeStruct((t, 1), jnp.float32),
            jax.ShapeDtypeStruct((t, 1), jnp.float32),
            jax.ShapeDtypeStruct((2 * t, 1), jnp.int32),
            jax.ShapeDtypeStruct((nt + 1, 1), jnp.int32),
        ],
        interpret=interpret,
    )(x, wg)


_CHUNK = 32


def _sc_scatter_rows(x, dest3, np_rows):
    t, d = x.shape
    nch = dest3.shape[1]
    per_w = nch * _CHUNK
    mesh = plsc.VectorSubcoreMesh(core_axis_name="c", subcore_axis_name="s")

    @functools.partial(
        pl.kernel, mesh=mesh,
        out_type=jax.ShapeDtypeStruct((np_rows, d), jnp.float32),
        scratch_types=[
            pltpu.VMEM((nch, _CHUNK), jnp.int32),
            pltpu.VMEM((2, _CHUNK, d), jnp.float32),
            pltpu.SemaphoreType.DMA,
            pltpu.SemaphoreType.DMA,
            pltpu.SemaphoreType.DMA,
            pltpu.SemaphoreType.DMA,
        ],
    )
    def k(x_hbm, dest_hbm, xs_hbm, idx_all, rows2, sr0, sr1, ss0, ss1):
        wid = lax.axis_index("s") * _NC + lax.axis_index("c")
        pltpu.sync_copy(dest_hbm.at[wid], idx_all)
        semr = (sr0, sr1)
        sems = (ss0, ss1)
        reads = [None, None]
        scats = [None, None]

        def issue_read(c):
            b = c % 2
            j0 = wid * per_w + c * _CHUNK
            r0 = lax.rem(j0, t)
            reads[b] = pltpu.async_copy(
                x_hbm.at[pl.ds(r0, _CHUNK)], rows2.at[b], semr[b])

        issue_read(0)
        for c in range(nch):
            b = c % 2
            if c + 1 < nch:
                if scats[(c + 1) % 2] is not None:
                    scats[(c + 1) % 2].wait()
                    scats[(c + 1) % 2] = None
                issue_read(c + 1)
            reads[b].wait()
            scats[b] = pltpu.async_copy(
                rows2.at[b], xs_hbm.at[idx_all.at[c]], sems[b])
        for b in range(2):
            if scats[b] is not None:
                scats[b].wait()

    return k(x, dest3)


def _sc_gather_rows(ys, dest3, n_tasks):
    d = ys.shape[1]
    nch = dest3.shape[1]
    per_w = nch * _CHUNK
    mesh = plsc.VectorSubcoreMesh(core_axis_name="c", subcore_axis_name="s")

    @functools.partial(
        pl.kernel, mesh=mesh,
        out_type=jax.ShapeDtypeStruct((n_tasks, d), jnp.float32),
        scratch_types=[
            pltpu.VMEM((nch, _CHUNK), jnp.int32),
            pltpu.VMEM((2, _CHUNK, d), jnp.float32),
            pltpu.SemaphoreType.DMA,
            pltpu.SemaphoreType.DMA,
            pltpu.SemaphoreType.DMA,
            pltpu.SemaphoreType.DMA,
        ],
    )
    def k(ys_hbm, dest_hbm, yg_hbm, idx_all, rows2, sg0, sg1, sw0, sw1):
        wid = lax.axis_index("s") * _NC + lax.axis_index("c")
        pltpu.sync_copy(dest_hbm.at[wid], idx_all)
        semg = (sg0, sg1)
        semw = (sw0, sw1)
        gats = [None, None]
        wrs = [None, None]

        def issue_gather(c):
            b = c % 2
            gats[b] = pltpu.async_copy(
                ys_hbm.at[idx_all.at[c]], rows2.at[b], semg[b])

        issue_gather(0)
        for c in range(nch):
            b = c % 2
            if c + 1 < nch:
                if wrs[(c + 1) % 2] is not None:
                    wrs[(c + 1) % 2].wait()
                    wrs[(c + 1) % 2] = None
                issue_gather(c + 1)
            gats[b].wait()
            j0 = wid * per_w + c * _CHUNK
            wrs[b] = pltpu.async_copy(
                rows2.at[b], yg_hbm.at[pl.ds(j0, _CHUNK)], semw[b])
        for b in range(2):
            if wrs[b] is not None:
                wrs[b].wait()

    return k(ys, dest3)


def _gmlp_body(te_ref, xs_ref, w1_ref, w3_ref, w2_ref, ys_ref, *, nt):
    i = pl.program_id(0)

    @pl.when(i < te_ref[nt])
    def _compute():
        h = xs_ref[...]
        a = lax.dot_general(h, w1_ref[0], (((1,), (1,)), ((), ())),
                            preferred_element_type=jnp.float32)
        a = a * (1.0 / (1.0 + jnp.exp(-a)))
        b = lax.dot_general(h, w3_ref[0], (((1,), (1,)), ((), ())),
                            preferred_element_type=jnp.float32)
        y = lax.dot_general(a * b, w2_ref[0], (((1,), (1,)), ((), ())),
                            preferred_element_type=jnp.float32)
        ys_ref[...] = y


def _gmlp(te, xs, w1, w3, w2, *, interpret=False):
    np_rows, d = xs.shape
    ne, ff, _ = w1.shape
    nt = np_rows // TILE
    body = functools.partial(_gmlp_body, nt=nt)
    grid_spec = pltpu.PrefetchScalarGridSpec(
        num_scalar_prefetch=1,
        grid=(nt,),
        in_specs=[
            pl.BlockSpec((TILE, d), lambda i, te_r: (i, 0)),
            pl.BlockSpec((1, ff, d), lambda i, te_r: (te_r[i], 0, 0)),
            pl.BlockSpec((1, ff, d), lambda i, te_r: (te_r[i], 0, 0)),
            pl.BlockSpec((1, d, ff), lambda i, te_r: (te_r[i], 0, 0)),
        ],
        out_specs=pl.BlockSpec((TILE, d), lambda i, te_r: (i, 0)),
    )
    return pl.pallas_call(
        body,
        grid_spec=grid_spec,
        out_shape=jax.ShapeDtypeStruct((np_rows, d), jnp.float32),
        compiler_params=pltpu.CompilerParams(
            dimension_semantics=("arbitrary",),
        ),
        interpret=interpret,
    )(te, xs, w1, w3, w2)


def _combine_body(w1_ref, w2_ref, y1_ref, y2_ref, out_ref):
    out_ref[...] = y1_ref[...] * w1_ref[...] + y2_ref[...] * w2_ref[...]


def _combine(w1n, w2n, yg, t, *, interpret=False):
    d = yg.shape[1]
    rows = min(1024, t)
    nb = t // rows
    return pl.pallas_call(
        _combine_body,
        grid=(nb,),
        in_specs=[
            pl.BlockSpec((rows, 1), lambda r: (r, 0)),
            pl.BlockSpec((rows, 1), lambda r: (r, 0)),
            pl.BlockSpec((rows, d), lambda r: (r, 0)),
            pl.BlockSpec((rows, d), lambda r, _nb=nb: (_nb + r, 0)),
        ],
        out_specs=pl.BlockSpec((rows, d), lambda r: (r, 0)),
        out_shape=jax.ShapeDtypeStruct((t, d), jnp.float32),
        interpret=interpret,
    )(w1n, w2n, yg, yg)


def kernel(hidden_states, Wg, W1, W3, W2):
    bsz, seq, d = hidden_states.shape
    ne = Wg.shape[0]
    x = hidden_states.reshape(-1, d)
    t = x.shape[0]
    np_rows = 2 * t + ne * TILE
    nt = np_rows // TILE

    w1n, w2n, dest, te = _route_plan(x, Wg, nt)
    dest3 = dest.reshape(_NW, (2 * t) // (_NW * _CHUNK), _CHUNK)
    xs = _sc_scatter_rows(x, dest3, np_rows)
    ys = _gmlp(te.reshape(-1), xs, W1, W3, W2)
    yg = _sc_gather_rows(ys, dest3, 2 * t)
    out = _combine(w1n, w2n, yg, t)
    return out.reshape(bsz, seq, d)


# speedup vs baseline: 1.0898x; 1.0015x over previous
import functools

import jax
import jax.numpy as jnp
from jax import lax
from jax.experimental import pallas as pl
from jax.experimental.pallas import tpu as pltpu
from jax.experimental.pallas import tpu_sc as plsc

TILE = 256
_NC, _NS = 2, 16
_NW = _NC * _NS


def _scan_rows(c, t):
    k = 1
    while k < t:
        pad = jnp.zeros((k,) + c.shape[1:], c.dtype)
        c = c + jnp.concatenate([pad, c[:-k]], axis=0)
        k *= 2
    return c


def _scan_lanes(c, n):
    k = 1
    while k < n:
        pad = jnp.zeros(c.shape[:1] + (k,), c.dtype)
        c = c + jnp.concatenate([pad, c[:, :-k]], axis=1)
        k *= 2
    return c


def _route_plan_body(x_ref, wg_ref, w1o_ref, w2o_ref, dest_ref, te_ref,
                     *, t, ne, nt):
    x = x_ref[...]
    logits = lax.dot_general(x, wg_ref[...], (((1,), (1,)), ((), ())),
                             preferred_element_type=jnp.float32)
    m = jnp.max(logits, axis=1, keepdims=True)
    ex = jnp.exp(logits - m)
    p = ex / jnp.sum(ex, axis=1, keepdims=True)
    iota = lax.broadcasted_iota(jnp.int32, p.shape, 1)
    v1 = jnp.max(p, axis=1, keepdims=True)
    e1 = jnp.min(jnp.where(p >= v1, iota, ne), axis=1, keepdims=True)
    pm = jnp.where(iota == e1, -1.0, p)
    v2 = jnp.max(pm, axis=1, keepdims=True)
    e2 = jnp.min(jnp.where(pm >= v2, iota, ne), axis=1, keepdims=True)
    s = v1 + v2
    w1o_ref[...] = v1 / s
    w2o_ref[...] = v2 / s
    io_e1 = lax.broadcasted_iota(jnp.int32, (t, ne), 1)
    oh1 = (e1 == io_e1).astype(jnp.int32)
    oh2 = (e2 == io_e1).astype(jnp.int32)
    c1 = _scan_rows(oh1, t)
    c2 = _scan_rows(oh2, t)
    ex1 = c1 - oh1
    ex2 = c2 - oh2
    cnt1 = c1[t - 1:t, :]
    cnt = cnt1 + c2[t - 1:t, :]
    pc = ((cnt + (TILE - 1)) // TILE) * TILE
    off = _scan_lanes(pc, ne) - pc
    rank1 = jnp.sum(oh1 * ex1, axis=1, keepdims=True)
    rank2 = jnp.sum(oh2 * (cnt1 + ex2), axis=1, keepdims=True)
    base1 = jnp.sum(oh1 * off, axis=1, keepdims=True)
    base2 = jnp.sum(oh2 * off, axis=1, keepdims=True)
    dest_ref[...] = jnp.concatenate([base1 + rank1, base2 + rank2], axis=0)
    endc = off + pc
    tid = lax.broadcasted_iota(jnp.int32, (nt + 1, ne), 0)
    te = jnp.sum((tid * TILE >= endc).astype(jnp.int32), axis=1, keepdims=True)
    nlive = jnp.sum(pc, axis=1, keepdims=True) // TILE
    te_map = jnp.minimum(te[:nt], ne - 1)
    te_ref[...] = jnp.concatenate([te_map, nlive], axis=0)


def _route_plan(x, wg, nt, *, interpret=False):
    t, d = x.shape
    ne = wg.shape[0]
    body = functools.partial(_route_plan_body, t=t, ne=ne, nt=nt)
    return pl.pallas_call(
        body,
        out_shape=[
            jax.ShapeDtypeStruct((t, 1), jnp.float32),
            jax.ShapeDtypeStruct((t, 1), jnp.float32),
            jax.ShapeDtypeStruct((2 * t, 1), jnp.int32),
            jax.ShapeDtypeStruct((nt + 1, 1), jnp.int32),
        ],
        interpret=interpret,
    )(x, wg)


_CHUNK = 32


def _sc_scatter_rows(x, dest3, np_rows):
    t, d = x.shape
    nch = dest3.shape[1]
    per_w = nch * _CHUNK
    mesh = plsc.VectorSubcoreMesh(core_axis_name="c", subcore_axis_name="s")

    @functools.partial(
        pl.kernel, mesh=mesh,
        out_type=jax.ShapeDtypeStruct((np_rows, d), jnp.float32),
        scratch_types=[
            pltpu.VMEM((nch, _CHUNK), jnp.int32),
            pltpu.VMEM((2, _CHUNK, d), jnp.float32),
            pltpu.SemaphoreType.DMA,
            pltpu.SemaphoreType.DMA,
            pltpu.SemaphoreType.DMA,
            pltpu.SemaphoreType.DMA,
        ],
    )
    def k(x_hbm, dest_hbm, xs_hbm, idx_all, rows2, sr0, sr1, ss0, ss1):
        wid = lax.axis_index("s") * _NC + lax.axis_index("c")
        pltpu.sync_copy(dest_hbm.at[wid], idx_all)
        semr = (sr0, sr1)
        sems = (ss0, ss1)
        reads = [None, None]
        scats = [None, None]

        def issue_read(c):
            b = c % 2
            j0 = wid * per_w + c * _CHUNK
            r0 = lax.rem(j0, t)
            reads[b] = pltpu.async_copy(
                x_hbm.at[pl.ds(r0, _CHUNK)], rows2.at[b], semr[b])

        issue_read(0)
        for c in range(nch):
            b = c % 2
            if c + 1 < nch:
                if scats[(c + 1) % 2] is not None:
                    scats[(c + 1) % 2].wait()
                    scats[(c + 1) % 2] = None
                issue_read(c + 1)
            reads[b].wait()
            scats[b] = pltpu.async_copy(
                rows2.at[b], xs_hbm.at[idx_all.at[c]], sems[b])
        for b in range(2):
            if scats[b] is not None:
                scats[b].wait()

    return k(x, dest3)


def _sc_gather_rows(ys, dest3, n_tasks):
    d = ys.shape[1]
    nch = dest3.shape[1]
    per_w = nch * _CHUNK
    mesh = plsc.VectorSubcoreMesh(core_axis_name="c", subcore_axis_name="s")

    @functools.partial(
        pl.kernel, mesh=mesh,
        out_type=jax.ShapeDtypeStruct((n_tasks, d), jnp.float32),
        scratch_types=[
            pltpu.VMEM((nch, _CHUNK), jnp.int32),
            pltpu.VMEM((2, _CHUNK, d), jnp.float32),
            pltpu.SemaphoreType.DMA,
            pltpu.SemaphoreType.DMA,
            pltpu.SemaphoreType.DMA,
            pltpu.SemaphoreType.DMA,
        ],
    )
    def k(ys_hbm, dest_hbm, yg_hbm, idx_all, rows2, sg0, sg1, sw0, sw1):
        wid = lax.axis_index("s") * _NC + lax.axis_index("c")
        pltpu.sync_copy(dest_hbm.at[wid], idx_all)
        semg = (sg0, sg1)
        semw = (sw0, sw1)
        gats = [None, None]
        wrs = [None, None]

        def issue_gather(c):
            b = c % 2
            gats[b] = pltpu.async_copy(
                ys_hbm.at[idx_all.at[c]], rows2.at[b], semg[b])

        issue_gather(0)
        for c in range(nch):
            b = c % 2
            if c + 1 < nch:
                if wrs[(c + 1) % 2] is not None:
                    wrs[(c + 1) % 2].wait()
                    wrs[(c + 1) % 2] = None
                issue_gather(c + 1)
            gats[b].wait()
            j0 = wid * per_w + c * _CHUNK
            wrs[b] = pltpu.async_copy(
                rows2.at[b], yg_hbm.at[pl.ds(j0, _CHUNK)], semw[b])
        for b in range(2):
            if wrs[b] is not None:
                wrs[b].wait()

    return k(ys, dest3)


def _gmlp_body(te_ref, xs_ref, w1_ref, w3_ref, w2_ref, ys_ref, *, nt):
    i = pl.program_id(0)

    @pl.when(i < te_ref[nt])
    def _compute():
        h = xs_ref[...]
        a = lax.dot_general(h, w1_ref[0], (((1,), (1,)), ((), ())),
                            preferred_element_type=jnp.float32)
        a = a * (1.0 / (1.0 + jnp.exp(-a)))
        b = lax.dot_general(h, w3_ref[0], (((1,), (1,)), ((), ())),
                            preferred_element_type=jnp.float32)
        y = lax.dot_general(a * b, w2_ref[0], (((1,), (1,)), ((), ())),
                            preferred_element_type=jnp.float32)
        ys_ref[...] = y


def _gmlp(te, xs, w1, w3, w2, *, interpret=False):
    np_rows, d = xs.shape
    ne, ff, _ = w1.shape
    nt = np_rows // TILE
    body = functools.partial(_gmlp_body, nt=nt)
    grid_spec = pltpu.PrefetchScalarGridSpec(
        num_scalar_prefetch=1,
        grid=(nt,),
        in_specs=[
            pl.BlockSpec((TILE, d), lambda i, te_r: (i, 0)),
            pl.BlockSpec((1, ff, d), lambda i, te_r: (te_r[i], 0, 0)),
            pl.BlockSpec((1, ff, d), lambda i, te_r: (te_r[i], 0, 0)),
            pl.BlockSpec((1, d, ff), lambda i, te_r: (te_r[i], 0, 0)),
        ],
        out_specs=pl.BlockSpec((TILE, d), lambda i, te_r: (i, 0)),
    )
    return pl.pallas_call(
        body,
        grid_spec=grid_spec,
        out_shape=jax.ShapeDtypeStruct((np_rows, d), jnp.float32),
        compiler_params=pltpu.CompilerParams(
            dimension_semantics=("arbitrary",),
        ),
        interpret=interpret,
    )(te, xs, w1, w3, w2)


def _combine_body(w1_ref, w2_ref, y1_ref, y2_ref, out_ref):
    out_ref[...] = y1_ref[...] * w1_ref[...] + y2_ref[...] * w2_ref[...]


def _combine(w1n, w2n, yg, t, *, interpret=False):
    d = yg.shape[1]
    rows = min(1024, t)
    nb = t // rows
    return pl.pallas_call(
        _combine_body,
        grid=(nb,),
        in_specs=[
            pl.BlockSpec((rows, 1), lambda r: (r, 0)),
            pl.BlockSpec((rows, 1), lambda r: (r, 0)),
            pl.BlockSpec((rows, d), lambda r: (r, 0)),
            pl.BlockSpec((rows, d), lambda r, _nb=nb: (_nb + r, 0)),
        ],
        out_specs=pl.BlockSpec((rows, d), lambda r: (r, 0)),
        out_shape=jax.ShapeDtypeStruct((t, d), jnp.float32),
        interpret=interpret,
    )(w1n, w2n, yg, yg)


def kernel(hidden_states, Wg, W1, W3, W2):
    bsz, seq, d = hidden_states.shape
    ne = Wg.shape[0]
    x = hidden_states.reshape(-1, d)
    t = x.shape[0]
    np_rows = 2 * t + ne * TILE
    nt = np_rows // TILE

    w1n, w2n, dest, te = _route_plan(x, Wg, nt)
    dest3 = dest.reshape(_NW, (2 * t) // (_NW * _CHUNK), _CHUNK)
    xs = _sc_scatter_rows(x, dest3, np_rows)
    ys = _gmlp(te.reshape(-1), xs, W1, W3, W2)
    yg = _sc_gather_rows(ys, dest3, 2 * t)
    out = _combine(w1n, w2n, yg, t)
    return out.reshape(bsz, seq, d)
